# baseline (speedup 1.0000x reference)
import functools

import jax
import jax.numpy as jnp
from jax import lax
from jax.experimental import pallas as pl
from jax.experimental.pallas import tpu as pltpu
from jax.experimental.pallas import tpu_sc as plsc

B = 1024
L = 50
D = 512
T = L + 2

_INFO = plsc.get_sparse_core_info()
_NC = _INFO.num_cores
_NS = _INFO.num_subcores
_NW = _NC * _NS
_BPW = B // _NW


def _body(feat_hbm, gidx_hbm, didx_hbm, wg_hbm, wd_hbm, out_hbm,
          gidx_v, didx_v, rows_g, rows_d, sem_g, sem_d):
    wid = lax.axis_index("s") * _NC + lax.axis_index("c")
    base = wid * _BPW

    pltpu.sync_copy(gidx_hbm.at[pl.ds(base, _BPW)], gidx_v)
    pltpu.sync_copy(didx_hbm.at[pl.ds(base, _BPW)], didx_v)

    cp_g = pltpu.async_copy(wg_hbm.at[gidx_v], rows_g, sem_g)
    cp_d = pltpu.async_copy(wd_hbm.at[didx_v], rows_d, sem_d)
    cp_g.wait()
    cp_d.wait()

    pltpu.sync_copy(rows_g, out_hbm.at[pl.ds(base, _BPW), 0])
    pltpu.sync_copy(rows_d, out_hbm.at[pl.ds(base, _BPW), 1])
    pltpu.sync_copy(feat_hbm.at[pl.ds(base, _BPW)],
                    out_hbm.at[pl.ds(base, _BPW), pl.ds(2, L)])


@jax.jit
def _run(feature, genre_index, difficulty_index, W_genre, W_difficulty):
    mesh = plsc.VectorSubcoreMesh(core_axis_name="c", subcore_axis_name="s")
    fn = pl.kernel(
        _body,
        out_type=jax.ShapeDtypeStruct((B, T, D), jnp.float32),
        mesh=mesh,
        scratch_types=[
            pltpu.VMEM((_BPW,), jnp.int32),
            pltpu.VMEM((_BPW,), jnp.int32),
            pltpu.VMEM((_BPW, D), jnp.float32),
            pltpu.VMEM((_BPW, D), jnp.float32),
            pltpu.SemaphoreType.DMA,
            pltpu.SemaphoreType.DMA,
        ],
        compiler_params=pltpu.CompilerParams(use_tc_tiling_on_sc=False),
    )
    return fn(feature, genre_index, difficulty_index, W_genre, W_difficulty)


def kernel(feature, genre_index, difficulty_index, W_genre, W_difficulty):
    gidx = genre_index.reshape(B).astype(jnp.int32)
    didx = difficulty_index.reshape(B).astype(jnp.int32)
    return _run(feature, gidx, didx, W_genre, W_difficulty)


# speedup vs baseline: 1.0021x; 1.0021x over previous
import functools

import jax
import jax.numpy as jnp
from jax import lax
from jax.experimental import pallas as pl
from jax.experimental.pallas import tpu as pltpu
from jax.experimental.pallas import tpu_sc as plsc

B = 1024
L = 50
D = 512
T = L + 2

_INFO = plsc.get_sparse_core_info()
_NC = _INFO.num_cores
_NS = _INFO.num_subcores
_NW = _NC * _NS
_BPW = B // _NW


def _body(feat_hbm, gidx_hbm, didx_hbm, wg_hbm, wd_hbm, out_hbm,
          gidx_v, didx_v, rows_g, rows_d, sem_g, sem_d, sem_f):
    wid = lax.axis_index("s") * _NC + lax.axis_index("c")
    base = wid * _BPW

    pltpu.sync_copy(gidx_hbm.at[pl.ds(base, _BPW)], gidx_v)
    pltpu.sync_copy(didx_hbm.at[pl.ds(base, _BPW)], didx_v)

    cp_g = pltpu.async_copy(wg_hbm.at[gidx_v], rows_g, sem_g)
    cp_d = pltpu.async_copy(wd_hbm.at[didx_v], rows_d, sem_d)

    feat_cps = []
    for j in range(_BPW):
        b = base + j
        feat_cps.append(
            pltpu.async_copy(feat_hbm.at[b], out_hbm.at[b, pl.ds(2, L)],
                             sem_f))
    cp_g.wait()
    cp_d.wait()
    wr_g = pltpu.async_copy(rows_g, out_hbm.at[pl.ds(base, _BPW), 0], sem_g)
    wr_d = pltpu.async_copy(rows_d, out_hbm.at[pl.ds(base, _BPW), 1], sem_d)
    for cp in feat_cps:
        cp.wait()
    wr_g.wait()
    wr_d.wait()


@jax.jit
def _run(feature, genre_index, difficulty_index, W_genre, W_difficulty):
    mesh = plsc.VectorSubcoreMesh(core_axis_name="c", subcore_axis_name="s")
    fn = pl.kernel(
        _body,
        out_type=jax.ShapeDtypeStruct((B, T, D), jnp.float32),
        mesh=mesh,
        scratch_types=[
            pltpu.VMEM((_BPW,), jnp.int32),
            pltpu.VMEM((_BPW,), jnp.int32),
            pltpu.VMEM((_BPW, D), jnp.float32),
            pltpu.VMEM((_BPW, D), jnp.float32),
            pltpu.SemaphoreType.DMA,
            pltpu.SemaphoreType.DMA,
            pltpu.SemaphoreType.DMA,
        ],
        compiler_params=pltpu.CompilerParams(use_tc_tiling_on_sc=False),
    )
    return fn(feature, genre_index, difficulty_index, W_genre, W_difficulty)


def kernel(feature, genre_index, difficulty_index, W_genre, W_difficulty):
    gidx = genre_index.reshape(B).astype(jnp.int32)
    didx = difficulty_index.reshape(B).astype(jnp.int32)
    return _run(feature, gidx, didx, W_genre, W_difficulty)


# speedup vs baseline: 7.6475x; 7.6312x over previous
import functools

import jax
import jax.numpy as jnp
from jax import lax
from jax.experimental import pallas as pl
from jax.experimental.pallas import tpu as pltpu
from jax.experimental.pallas import tpu_sc as plsc

B = 1024
L = 50
D = 512
T = L + 2

_INFO = plsc.get_sparse_core_info()
_NC = _INFO.num_cores
_NS = _INFO.num_subcores
_NW = _NC * _NS
_BPW = B // _NW


_NBUF = 3


def _body(feat_hbm, gidx_hbm, didx_hbm, wg_hbm, wd_hbm, out_hbm,
          gidx_v, didx_v, rows_g, rows_d, fbuf,
          sem_g, sem_d, sems_in, sems_out):
    wid = lax.axis_index("s") * _NC + lax.axis_index("c")
    base = wid * _BPW

    pltpu.sync_copy(gidx_hbm.at[pl.ds(base, _BPW)], gidx_v)
    pltpu.sync_copy(didx_hbm.at[pl.ds(base, _BPW)], didx_v)

    cp_g = pltpu.async_copy(wg_hbm.at[gidx_v], rows_g, sem_g)
    cp_d = pltpu.async_copy(wd_hbm.at[didx_v], rows_d, sem_d)

    in_cps = [None] * _BPW
    out_cps = [None] * _BPW
    for j in range(_NBUF):
        in_cps[j] = pltpu.async_copy(feat_hbm.at[base + j], fbuf.at[j],
                                     sems_in[j])
    for j in range(_BPW):
        k = j % _NBUF
        in_cps[j].wait()
        out_cps[j] = pltpu.async_copy(fbuf.at[k],
                                      out_hbm.at[base + j, pl.ds(2, L)],
                                      sems_out[k])
        if j + _NBUF < _BPW:
            out_cps[j].wait()
            in_cps[j + _NBUF] = pltpu.async_copy(
                feat_hbm.at[base + j + _NBUF], fbuf.at[k], sems_in[k])

    cp_g.wait()
    cp_d.wait()
    wr_g = pltpu.async_copy(rows_g, out_hbm.at[pl.ds(base, _BPW), 0], sem_g)
    wr_d = pltpu.async_copy(rows_d, out_hbm.at[pl.ds(base, _BPW), 1], sem_d)
    for j in range(_BPW - _NBUF, _BPW):
        out_cps[j].wait()
    wr_g.wait()
    wr_d.wait()


@jax.jit
def _run(feature, genre_index, difficulty_index, W_genre, W_difficulty):
    mesh = plsc.VectorSubcoreMesh(core_axis_name="c", subcore_axis_name="s")
    fn = pl.kernel(
        _body,
        out_type=jax.ShapeDtypeStruct((B, T, D), jnp.float32),
        mesh=mesh,
        scratch_types=[
            pltpu.VMEM((_BPW,), jnp.int32),
            pltpu.VMEM((_BPW,), jnp.int32),
            pltpu.VMEM((_BPW, D), jnp.float32),
            pltpu.VMEM((_BPW, D), jnp.float32),
            pltpu.VMEM((_NBUF, L, D), jnp.float32),
            pltpu.SemaphoreType.DMA,
            pltpu.SemaphoreType.DMA,
            [pltpu.SemaphoreType.DMA] * _NBUF,
            [pltpu.SemaphoreType.DMA] * _NBUF,
        ],
        compiler_params=pltpu.CompilerParams(use_tc_tiling_on_sc=False),
    )
    return fn(feature, genre_index, difficulty_index, W_genre, W_difficulty)


def kernel(feature, genre_index, difficulty_index, W_genre, W_difficulty):
    gidx = genre_index.reshape(B).astype(jnp.int32)
    didx = difficulty_index.reshape(B).astype(jnp.int32)
    return _run(feature, gidx, didx, W_genre, W_difficulty)


# speedup vs baseline: 10.2067x; 1.3346x over previous
import functools

import jax
import jax.numpy as jnp
from jax import lax
from jax.experimental import pallas as pl
from jax.experimental.pallas import tpu as pltpu
from jax.experimental.pallas import tpu_sc as plsc

B = 1024
L = 50
D = 512
T = L + 2

_INFO = plsc.get_sparse_core_info()
_NC = _INFO.num_cores
_NS = _INFO.num_subcores
_NW = _NC * _NS
_BPW = B // _NW


def _gather_body(gidx_hbm, didx_hbm, wg_hbm, wd_hbm, eg_hbm, ed_hbm,
                 gidx_v, didx_v, rows_g, rows_d, sem_g, sem_d):
    wid = lax.axis_index("s") * _NC + lax.axis_index("c")
    base = wid * _BPW

    pltpu.sync_copy(gidx_hbm.at[pl.ds(base, _BPW)], gidx_v)
    pltpu.sync_copy(didx_hbm.at[pl.ds(base, _BPW)], didx_v)

    cp_g = pltpu.async_copy(wg_hbm.at[gidx_v], rows_g, sem_g)
    cp_d = pltpu.async_copy(wd_hbm.at[didx_v], rows_d, sem_d)
    cp_g.wait()
    cp_d.wait()
    wr_g = pltpu.async_copy(rows_g, eg_hbm.at[pl.ds(base, _BPW)], sem_g)
    wr_d = pltpu.async_copy(rows_d, ed_hbm.at[pl.ds(base, _BPW)], sem_d)
    wr_g.wait()
    wr_d.wait()


def _assemble_body(f_ref, eg_ref, ed_ref, out_ref):
    out_ref[:, 0, :] = eg_ref[...]
    out_ref[:, 1, :] = ed_ref[...]
    out_ref[:, 2:, :] = f_ref[...]


_BB = 8


@jax.jit
def _run(feature, genre_index, difficulty_index, W_genre, W_difficulty):
    mesh = plsc.VectorSubcoreMesh(core_axis_name="c", subcore_axis_name="s")
    gather = pl.kernel(
        _gather_body,
        out_type=(jax.ShapeDtypeStruct((B, D), jnp.float32),
                  jax.ShapeDtypeStruct((B, D), jnp.float32)),
        mesh=mesh,
        scratch_types=[
            pltpu.VMEM((_BPW,), jnp.int32),
            pltpu.VMEM((_BPW,), jnp.int32),
            pltpu.VMEM((_BPW, D), jnp.float32),
            pltpu.VMEM((_BPW, D), jnp.float32),
            pltpu.SemaphoreType.DMA,
            pltpu.SemaphoreType.DMA,
        ],
    )
    eg, ed = gather(genre_index, difficulty_index, W_genre, W_difficulty)

    assemble = pl.pallas_call(
        _assemble_body,
        grid=(B // _BB,),
        in_specs=[
            pl.BlockSpec((_BB, L, D), lambda b: (b, 0, 0)),
            pl.BlockSpec((_BB, D), lambda b: (b, 0)),
            pl.BlockSpec((_BB, D), lambda b: (b, 0)),
        ],
        out_specs=pl.BlockSpec((_BB, T, D), lambda b: (b, 0, 0)),
        out_shape=jax.ShapeDtypeStruct((B, T, D), jnp.float32),
        compiler_params=pltpu.CompilerParams(
            dimension_semantics=("arbitrary",)),
    )
    return assemble(feature, eg, ed)


def kernel(feature, genre_index, difficulty_index, W_genre, W_difficulty):
    gidx = genre_index.reshape(B).astype(jnp.int32)
    didx = difficulty_index.reshape(B).astype(jnp.int32)
    return _run(feature, gidx, didx, W_genre, W_difficulty)


# speedup vs baseline: 11.7537x; 1.1516x over previous
import functools

import jax
import jax.numpy as jnp
from jax import lax
from jax.experimental import pallas as pl
from jax.experimental.pallas import tpu as pltpu
from jax.experimental.pallas import tpu_sc as plsc

B = 1024
L = 50
D = 512
T = L + 2

_INFO = plsc.get_sparse_core_info()
_NC = _INFO.num_cores
_NS = _INFO.num_subcores
_NW = _NS
_BPW = B // _NW


def _gather_body(gidx_hbm, didx_hbm, wg_hbm, wd_hbm, eg_hbm, ed_hbm,
                 gidx_v, didx_v, rows_g, rows_d, sem_g, sem_d):
    wid = lax.axis_index("s")
    base = wid * _BPW

    pltpu.sync_copy(gidx_hbm.at[pl.ds(base, _BPW)], gidx_v)
    pltpu.sync_copy(didx_hbm.at[pl.ds(base, _BPW)], didx_v)

    cp_g = pltpu.async_copy(wg_hbm.at[gidx_v], rows_g, sem_g)
    cp_d = pltpu.async_copy(wd_hbm.at[didx_v], rows_d, sem_d)
    cp_g.wait()
    cp_d.wait()
    wr_g = pltpu.async_copy(rows_g, eg_hbm.at[pl.ds(base, _BPW)], sem_g)
    wr_d = pltpu.async_copy(rows_d, ed_hbm.at[pl.ds(base, _BPW)], sem_d)
    wr_g.wait()
    wr_d.wait()


def _assemble_body(f_ref, eg_ref, ed_ref, out_ref):
    out_ref[:, 0, :] = eg_ref[...]
    out_ref[:, 1, :] = ed_ref[...]
    out_ref[:, 2:, :] = f_ref[...]


_BB = 32


@jax.jit
def _run(feature, genre_index, difficulty_index, W_genre, W_difficulty):
    mesh = plsc.VectorSubcoreMesh(core_axis_name="c", subcore_axis_name="s", num_cores=1)
    gather = pl.kernel(
        _gather_body,
        out_type=(jax.ShapeDtypeStruct((B, D), jnp.float32),
                  jax.ShapeDtypeStruct((B, D), jnp.float32)),
        mesh=mesh,
        scratch_types=[
            pltpu.VMEM((_BPW,), jnp.int32),
            pltpu.VMEM((_BPW,), jnp.int32),
            pltpu.VMEM((_BPW, D), jnp.float32),
            pltpu.VMEM((_BPW, D), jnp.float32),
            pltpu.SemaphoreType.DMA,
            pltpu.SemaphoreType.DMA,
        ],
    )
    eg, ed = gather(genre_index, difficulty_index, W_genre, W_difficulty)

    assemble = pl.pallas_call(
        _assemble_body,
        grid=(B // _BB,),
        in_specs=[
            pl.BlockSpec((_BB, L, D), lambda b: (b, 0, 0)),
            pl.BlockSpec((_BB, D), lambda b: (b, 0)),
            pl.BlockSpec((_BB, D), lambda b: (b, 0)),
        ],
        out_specs=pl.BlockSpec((_BB, T, D), lambda b: (b, 0, 0)),
        out_shape=jax.ShapeDtypeStruct((B, T, D), jnp.float32),
        compiler_params=pltpu.CompilerParams(
            dimension_semantics=("parallel",)),
    )
    return assemble(feature, eg, ed)


def kernel(feature, genre_index, difficulty_index, W_genre, W_difficulty):
    gidx = genre_index.reshape(B).astype(jnp.int32)
    didx = difficulty_index.reshape(B).astype(jnp.int32)
    return _run(feature, gidx, didx, W_genre, W_difficulty)


# speedup vs baseline: 11.7809x; 1.0023x over previous
import functools

import jax
import jax.numpy as jnp
from jax import lax
from jax.experimental import pallas as pl
from jax.experimental.pallas import tpu as pltpu
from jax.experimental.pallas import tpu_sc as plsc

B = 1024
L = 50
D = 512
T = L + 2

_INFO = plsc.get_sparse_core_info()
_NC = _INFO.num_cores
_NS = _INFO.num_subcores
_NW = _NS
_BPW = B // _NW


def _gather_body(gidx_hbm, didx_hbm, wg_hbm, wd_hbm, eg_hbm, ed_hbm,
                 gidx_v, didx_v, rows_g, rows_d, sem_g, sem_d):
    wid = lax.axis_index("s")
    base = wid * _BPW

    pltpu.sync_copy(gidx_hbm.at[pl.ds(base, _BPW)], gidx_v)
    pltpu.sync_copy(didx_hbm.at[pl.ds(base, _BPW)], didx_v)

    cp_g = pltpu.async_copy(wg_hbm.at[gidx_v], rows_g, sem_g)
    cp_d = pltpu.async_copy(wd_hbm.at[didx_v], rows_d, sem_d)
    cp_g.wait()
    cp_d.wait()
    wr_g = pltpu.async_copy(rows_g, eg_hbm.at[pl.ds(base, _BPW)], sem_g)
    wr_d = pltpu.async_copy(rows_d, ed_hbm.at[pl.ds(base, _BPW)], sem_d)
    wr_g.wait()
    wr_d.wait()


_BB = 16
_NST = B // _BB
_CB = 3


def _assemble_body(f_hbm, eg_hbm, ed_hbm, out_hbm,
                   fbuf, gbuf, dbuf, obuf, sem_f, sem_g, sem_d, sem_o):
    def in_cps(i, k):
        row = i * _BB
        return (
            pltpu.make_async_copy(f_hbm.at[pl.ds(row, _BB)], fbuf.at[k],
                                  sem_f.at[k]),
            pltpu.make_async_copy(eg_hbm.at[pl.ds(row, _BB)], gbuf.at[k],
                                  sem_g.at[k]),
            pltpu.make_async_copy(ed_hbm.at[pl.ds(row, _BB)], dbuf.at[k],
                                  sem_d.at[k]),
        )

    def out_cp(i, k):
        return pltpu.make_async_copy(obuf.at[k], out_hbm.at[pl.ds(i * _BB, _BB)],
                                     sem_o.at[k])

    for k in range(_CB):
        for cp in in_cps(k, k):
            cp.start()

    def step(i, _):
        k = lax.rem(i, _CB)
        for cp in in_cps(i, k):
            cp.wait()

        @pl.when(i >= _CB)
        def _():
            out_cp(i - _CB, k).wait()

        obuf[k, :, 0, :] = gbuf[k]
        obuf[k, :, 1, :] = dbuf[k]
        obuf[k, :, 2:, :] = fbuf[k]
        out_cp(i, k).start()

        @pl.when(i + _CB < _NST)
        def _():
            for cp in in_cps(i + _CB, k):
                cp.start()
        return 0

    lax.fori_loop(0, _NST, step, 0)
    for j in range(_NST - _CB, _NST):
        out_cp(j, j % _CB).wait()


@jax.jit
def _run(feature, genre_index, difficulty_index, W_genre, W_difficulty):
    mesh = plsc.VectorSubcoreMesh(core_axis_name="c", subcore_axis_name="s", num_cores=1)
    gather = pl.kernel(
        _gather_body,
        out_type=(jax.ShapeDtypeStruct((B, D), jnp.float32),
                  jax.ShapeDtypeStruct((B, D), jnp.float32)),
        mesh=mesh,
        scratch_types=[
            pltpu.VMEM((_BPW,), jnp.int32),
            pltpu.VMEM((_BPW,), jnp.int32),
            pltpu.VMEM((_BPW, D), jnp.float32),
            pltpu.VMEM((_BPW, D), jnp.float32),
            pltpu.SemaphoreType.DMA,
            pltpu.SemaphoreType.DMA,
        ],
    )
    eg, ed = gather(genre_index, difficulty_index, W_genre, W_difficulty)

    assemble = pl.pallas_call(
        _assemble_body,
        in_specs=[
            pl.BlockSpec(memory_space=pl.ANY),
            pl.BlockSpec(memory_space=pl.ANY),
            pl.BlockSpec(memory_space=pl.ANY),
        ],
        out_specs=pl.BlockSpec(memory_space=pl.ANY),
        out_shape=jax.ShapeDtypeStruct((B, T, D), jnp.float32),
        scratch_shapes=[
            pltpu.VMEM((_CB, _BB, L, D), jnp.float32),
            pltpu.VMEM((_CB, _BB, D), jnp.float32),
            pltpu.VMEM((_CB, _BB, D), jnp.float32),
            pltpu.VMEM((_CB, _BB, T, D), jnp.float32),
            pltpu.SemaphoreType.DMA((_CB,)),
            pltpu.SemaphoreType.DMA((_CB,)),
            pltpu.SemaphoreType.DMA((_CB,)),
            pltpu.SemaphoreType.DMA((_CB,)),
        ],
    )
    return assemble(feature, eg, ed)


def kernel(feature, genre_index, difficulty_index, W_genre, W_difficulty):
    gidx = genre_index.reshape(B).astype(jnp.int32)
    didx = difficulty_index.reshape(B).astype(jnp.int32)
    return _run(feature, gidx, didx, W_genre, W_difficulty)


# speedup vs baseline: 11.7829x; 1.0002x over previous
import functools

import jax
import jax.numpy as jnp
from jax import lax
from jax.experimental import pallas as pl
from jax.experimental.pallas import tpu as pltpu
from jax.experimental.pallas import tpu_sc as plsc

B = 1024
L = 50
D = 512
T = L + 2

_INFO = plsc.get_sparse_core_info()
_NC = _INFO.num_cores
_NS = _INFO.num_subcores
_NW = _NS
_BPW = B // _NW


def _gather_body(gidx_hbm, didx_hbm, wg_hbm, wd_hbm, eg_hbm, ed_hbm,
                 gidx_v, didx_v, rows_g, rows_d, sem_g, sem_d):
    wid = lax.axis_index("s")
    base = wid * _BPW

    pltpu.sync_copy(gidx_hbm.at[pl.ds(base, _BPW)], gidx_v)
    pltpu.sync_copy(didx_hbm.at[pl.ds(base, _BPW)], didx_v)

    cp_g = pltpu.async_copy(wg_hbm.at[gidx_v], rows_g, sem_g)
    cp_d = pltpu.async_copy(wd_hbm.at[didx_v], rows_d, sem_d)
    cp_g.wait()
    cp_d.wait()
    wr_g = pltpu.async_copy(rows_g, eg_hbm.at[pl.ds(base, _BPW)], sem_g)
    wr_d = pltpu.async_copy(rows_d, ed_hbm.at[pl.ds(base, _BPW)], sem_d)
    wr_g.wait()
    wr_d.wait()


_BB = 16
_NST = B // _BB
_CB = 3


def _assemble_body(f_hbm, eg_hbm, ed_hbm, out_hbm,
                   fbuf, gbuf, dbuf, obuf, sem_f, sem_g, sem_d, sem_o):
    def in_cps(i, k):
        row = i * _BB
        return (
            pltpu.make_async_copy(f_hbm.at[pl.ds(row, _BB)], fbuf.at[k],
                                  sem_f.at[k]),
            pltpu.make_async_copy(eg_hbm.at[pl.ds(row, _BB)], gbuf.at[k],
                                  sem_g.at[k]),
            pltpu.make_async_copy(ed_hbm.at[pl.ds(row, _BB)], dbuf.at[k],
                                  sem_d.at[k]),
        )

    def out_cp(i, k):
        return pltpu.make_async_copy(obuf.at[k], out_hbm.at[pl.ds(i * _BB, _BB)],
                                     sem_o.at[k])

    for k in range(_CB):
        for cp in in_cps(k, k):
            cp.start()

    def step(i, _):
        k = lax.rem(i, _CB)
        for cp in in_cps(i, k):
            cp.wait()

        @pl.when(i >= _CB)
        def _():
            out_cp(i - _CB, k).wait()

        obuf[k, :, 0, :] = gbuf[k]
        obuf[k, :, 1, :] = dbuf[k]
        obuf[k, :, 2:, :] = fbuf[k]
        out_cp(i, k).start(priority=1)

        @pl.when(i + _CB < _NST)
        def _():
            for cp in in_cps(i + _CB, k):
                cp.start()
        return 0

    lax.fori_loop(0, _NST, step, 0)
    for j in range(_NST - _CB, _NST):
        out_cp(j, j % _CB).wait()


@jax.jit
def _run(feature, genre_index, difficulty_index, W_genre, W_difficulty):
    mesh = plsc.VectorSubcoreMesh(core_axis_name="c", subcore_axis_name="s", num_cores=1)
    gather = pl.kernel(
        _gather_body,
        out_type=(jax.ShapeDtypeStruct((B, D), jnp.float32),
                  jax.ShapeDtypeStruct((B, D), jnp.float32)),
        mesh=mesh,
        scratch_types=[
            pltpu.VMEM((_BPW,), jnp.int32),
            pltpu.VMEM((_BPW,), jnp.int32),
            pltpu.VMEM((_BPW, D), jnp.float32),
            pltpu.VMEM((_BPW, D), jnp.float32),
            pltpu.SemaphoreType.DMA,
            pltpu.SemaphoreType.DMA,
        ],
    )
    eg, ed = gather(genre_index, difficulty_index, W_genre, W_difficulty)

    assemble = pl.pallas_call(
        _assemble_body,
        in_specs=[
            pl.BlockSpec(memory_space=pl.ANY),
            pl.BlockSpec(memory_space=pl.ANY),
            pl.BlockSpec(memory_space=pl.ANY),
        ],
        out_specs=pl.BlockSpec(memory_space=pl.ANY),
        out_shape=jax.ShapeDtypeStruct((B, T, D), jnp.float32),
        scratch_shapes=[
            pltpu.VMEM((_CB, _BB, L, D), jnp.float32),
            pltpu.VMEM((_CB, _BB, D), jnp.float32),
            pltpu.VMEM((_CB, _BB, D), jnp.float32),
            pltpu.VMEM((_CB, _BB, T, D), jnp.float32),
            pltpu.SemaphoreType.DMA((_CB,)),
            pltpu.SemaphoreType.DMA((_CB,)),
            pltpu.SemaphoreType.DMA((_CB,)),
            pltpu.SemaphoreType.DMA((_CB,)),
        ],
    )
    return assemble(feature, eg, ed)


def kernel(feature, genre_index, difficulty_index, W_genre, W_difficulty):
    gidx = genre_index.reshape(B).astype(jnp.int32)
    didx = difficulty_index.reshape(B).astype(jnp.int32)
    return _run(feature, gidx, didx, W_genre, W_difficulty)
